# baseline (speedup 1.0000x reference)
import jax
import jax.numpy as jnp
from jax.experimental import pallas as pl
from jax.experimental.pallas import tpu as pltpu

_B, _L, _H, _C = 16, 4096, 512, 2
_CHUNK = 128
_NCH = _L // _CHUNK
_NWORK = _B * _NCH


def _plan_kernel(m1_ref, m2_ref, len_ref, csum_ref, wrow_ref, wchunk_ref):
    t = jax.lax.broadcasted_iota(jnp.int32, (1, _NWORK), 1)
    for i, m_ref in enumerate((m1_ref, m2_ref)):
        m = m_ref[...]
        mn = jnp.min(m, axis=1, keepdims=True)
        pos = jax.lax.broadcasted_iota(jnp.int32, m.shape, 1)
        am = jnp.min(jnp.where(m == mn, pos, _L), axis=1)
        eff = jnp.where(am == 0, _L, am)
        len_ref[i, :] = eff

        n = (eff + (_CHUNK - 1)) // _CHUNK
        cs2 = n.reshape(1, _B)
        for k in (1, 2, 4, 8):
            cs2 = cs2 + jnp.concatenate(
                [jnp.zeros((1, k), jnp.int32), cs2[:, :-k]], axis=1
            )
        csum = cs2[0]
        csum_ref[i, :] = csum

        row = jnp.zeros((1, _NWORK), jnp.int32)
        start = jnp.zeros((1, _NWORK), jnp.int32)
        nsel = jnp.zeros((1, _NWORK), jnp.int32)
        for bb in range(_B):
            s_b = (csum[bb] - n[bb]).reshape(1, 1)
            in_row = t >= s_b
            row = jnp.where(in_row, bb, row)
            start = jnp.where(in_row, s_b, start)
            nsel = jnp.where(in_row, n[bb].reshape(1, 1), nsel)
        wrow_ref[i, :] = row[0]
        wchunk_ref[i, :] = jnp.minimum(t - start, nsel - 1)[0]


def _pool_kernel(len_ref, csum_ref, wrow_ref, wchunk_ref, x1_hbm, x2_hbm,
                 w_ref, bias_ref, out_ref, acc1, acc2):
    neg = jnp.finfo(jnp.float32).min
    acc1[...] = jnp.full(acc1.shape, neg, jnp.float32)
    acc2[...] = jnp.full(acc2.shape, neg, jnp.float32)

    nmax = jnp.maximum(csum_ref[0, _B - 1], csum_ref[1, _B - 1])
    ngroups = (nmax + 3) // 4

    def _item(i, tt2):
        return wrow_ref[i, tt2], wchunk_ref[i, tt2]

    def inner(idx, *blks):
        (tt,) = idx
        for j, blk in enumerate(blks):
            i, off = divmod(j, 4)
            acc = acc1 if i == 0 else acc2
            b, c = _item(i, 4 * tt + off)
            eff = len_ref[i, b]
            nch = pl.cdiv(eff, _CHUNK)

            @pl.when(c + 1 < nch)
            def _():
                x = blk[0].reshape(_CHUNK // 8, 8, _H)
                acc[b] = jnp.maximum(acc[b], jnp.max(x, axis=0))

            @pl.when(c + 1 == nch)
            def _():
                x = blk[0]
                pos = c * _CHUNK + jax.lax.broadcasted_iota(jnp.int32, x.shape, 0)
                xm = jnp.where(pos < eff, x, neg).reshape(_CHUNK // 8, 8, _H)
                acc[b] = jnp.maximum(acc[b], jnp.max(xm, axis=0))

    def _mk_index_map(i, off):
        def index_map(tt):
            b, c = _item(i, 4 * tt + off)
            return (b, c, 0)
        return index_map

    def _spec(i, off):
        return pl.BlockSpec(
            (1, _CHUNK, _H),
            _mk_index_map(i, off),
            pipeline_mode=pl.Buffered(buffer_count=8, use_lookahead=True),
        )

    pipe = pltpu.emit_pipeline(
        inner,
        grid=(ngroups,),
        in_specs=[_spec(i, off) for i in (0, 1) for off in range(4)],
        _explicit_indices=True,
    )
    pipe(x1_hbm, x1_hbm, x1_hbm, x1_hbm, x2_hbm, x2_hbm, x2_hbm, x2_hbm)

    h1 = jnp.max(acc1[...], axis=1)
    h2 = jnp.max(acc2[...], axis=1)
    out_ref[...] = (
        jnp.dot(h1, w_ref[:_H, :], preferred_element_type=jnp.float32)
        + jnp.dot(h2, w_ref[_H:, :], preferred_element_type=jnp.float32)
        + bias_ref[0, :]
    )


def kernel(x1, x2, m1, m2, W, b):
    lens, csum, wrow, wchunk = pl.pallas_call(
        _plan_kernel,
        out_shape=[
            jax.ShapeDtypeStruct((2, _B), jnp.int32),
            jax.ShapeDtypeStruct((2, _B), jnp.int32),
            jax.ShapeDtypeStruct((2, _NWORK), jnp.int32),
            jax.ShapeDtypeStruct((2, _NWORK), jnp.int32),
        ],
    )(m1, m2)

    out = pl.pallas_call(
        _pool_kernel,
        in_specs=[
            pl.BlockSpec(memory_space=pltpu.SMEM),
            pl.BlockSpec(memory_space=pltpu.SMEM),
            pl.BlockSpec(memory_space=pltpu.SMEM),
            pl.BlockSpec(memory_space=pltpu.SMEM),
            pl.BlockSpec(memory_space=pl.ANY),
            pl.BlockSpec(memory_space=pl.ANY),
            pl.BlockSpec(memory_space=pltpu.VMEM),
            pl.BlockSpec(memory_space=pltpu.VMEM),
        ],
        out_specs=pl.BlockSpec(memory_space=pltpu.VMEM),
        out_shape=jax.ShapeDtypeStruct((_B, _C), jnp.float32),
        scratch_shapes=[
            pltpu.VMEM((_B, 8, _H), jnp.float32),
            pltpu.VMEM((_B, 8, _H), jnp.float32),
        ],
    )(lens, csum, wrow, wchunk, x1, x2, W, b.reshape(1, _C))
    return out


# speedup vs baseline: 1.0704x; 1.0704x over previous
import jax
import jax.numpy as jnp
from jax.experimental import pallas as pl
from jax.experimental.pallas import tpu as pltpu

_B, _L, _H, _C = 16, 4096, 512, 2
_CHUNK = 256
_NCH = _L // _CHUNK
_NWORK = _B * _NCH


def _pool_kernel(m1_ref, m2_ref, x1_hbm, x2_hbm, w_ref, bias_ref, out_ref,
                 acc1, acc2, plan_v, plan_s, sem):
    neg = jnp.finfo(jnp.float32).min
    acc1[...] = jnp.full(acc1.shape, neg, jnp.float32)
    acc2[...] = jnp.full(acc2.shape, neg, jnp.float32)

    t = jax.lax.broadcasted_iota(jnp.int32, (1, _NWORK), 1)
    csums = []
    for i, m_ref in enumerate((m1_ref, m2_ref)):
        m = m_ref[...]
        mn = jnp.min(m, axis=1, keepdims=True)
        pos = jax.lax.broadcasted_iota(jnp.int32, m.shape, 1)
        am = jnp.min(jnp.where(m == mn, pos, _L), axis=1)
        eff = jnp.where(am == 0, _L, am)

        n = (eff + (_CHUNK - 1)) // _CHUNK
        cs2 = n.reshape(1, _B)
        for k in (1, 2, 4, 8):
            cs2 = cs2 + jnp.concatenate(
                [jnp.zeros((1, k), jnp.int32), cs2[:, :-k]], axis=1
            )
        csum = cs2[0]
        csums.append(csum)

        row = jnp.zeros((1, _NWORK), jnp.int32)
        start = jnp.zeros((1, _NWORK), jnp.int32)
        nsel = jnp.zeros((1, _NWORK), jnp.int32)
        for bb in range(_B):
            s_b = (csum[bb] - n[bb]).reshape(1, 1)
            in_row = t >= s_b
            row = jnp.where(in_row, bb, row)
            start = jnp.where(in_row, s_b, start)
            nsel = jnp.where(in_row, n[bb].reshape(1, 1), nsel)

        lens_pad = jnp.concatenate(
            [eff.reshape(1, _B), csum.reshape(1, _B),
             jnp.zeros((1, _NWORK - 2 * _B), jnp.int32)], axis=1)
        plan_v[0 + i, :] = lens_pad[0]
        plan_v[2 + i, :] = row[0]
        plan_v[4 + i, :] = jnp.minimum(t - start, nsel - 1)[0]

    copy = pltpu.make_async_copy(plan_v, plan_s, sem)
    copy.start()
    copy.wait()

    nmax = jnp.maximum(plan_s[0, _B + _B - 1], plan_s[1, _B + _B - 1])
    npairs = (nmax + 1) // 2

    def _item(i, tt2):
        return plan_s[2 + i, tt2], plan_s[4 + i, tt2]

    def inner(idx, x1a_blk, x1b_blk, x2a_blk, x2b_blk):
        (tt,) = idx
        for i, blk, off, acc in (
            (0, x1a_blk, 0, acc1),
            (0, x1b_blk, 1, acc1),
            (1, x2a_blk, 0, acc2),
            (1, x2b_blk, 1, acc2),
        ):
            b, c = _item(i, 2 * tt + off)
            eff = plan_s[0 + i, b]
            nch = pl.cdiv(eff, _CHUNK)

            @pl.when(c + 1 < nch)
            def _():
                x = blk[0].reshape(_CHUNK // 8, 8, _H)
                acc[b] = jnp.maximum(acc[b], jnp.max(x, axis=0))

            @pl.when(c + 1 == nch)
            def _():
                x = blk[0]
                pos = c * _CHUNK + jax.lax.broadcasted_iota(jnp.int32, x.shape, 0)
                xm = jnp.where(pos < eff, x, neg).reshape(_CHUNK // 8, 8, _H)
                acc[b] = jnp.maximum(acc[b], jnp.max(xm, axis=0))

    def _mk_index_map(i, off):
        def index_map(tt):
            b, c = _item(i, 2 * tt + off)
            return (b, c, 0)
        return index_map

    def _spec(i, off):
        return pl.BlockSpec(
            (1, _CHUNK, _H),
            _mk_index_map(i, off),
            pipeline_mode=pl.Buffered(buffer_count=8, use_lookahead=True),
        )

    pipe = pltpu.emit_pipeline(
        inner,
        grid=(npairs,),
        in_specs=[_spec(0, 0), _spec(0, 1), _spec(1, 0), _spec(1, 1)],
        _explicit_indices=True,
    )
    pipe(x1_hbm, x1_hbm, x2_hbm, x2_hbm)

    h1 = jnp.max(acc1[...], axis=1)
    h2 = jnp.max(acc2[...], axis=1)
    out_ref[...] = (
        jnp.dot(h1, w_ref[:_H, :], preferred_element_type=jnp.float32)
        + jnp.dot(h2, w_ref[_H:, :], preferred_element_type=jnp.float32)
        + bias_ref[0, :]
    )


def kernel(x1, x2, m1, m2, W, b):
    out = pl.pallas_call(
        _pool_kernel,
        in_specs=[
            pl.BlockSpec(memory_space=pltpu.VMEM),
            pl.BlockSpec(memory_space=pltpu.VMEM),
            pl.BlockSpec(memory_space=pl.ANY),
            pl.BlockSpec(memory_space=pl.ANY),
            pl.BlockSpec(memory_space=pltpu.VMEM),
            pl.BlockSpec(memory_space=pltpu.VMEM),
        ],
        out_specs=pl.BlockSpec(memory_space=pltpu.VMEM),
        out_shape=jax.ShapeDtypeStruct((_B, _C), jnp.float32),
        scratch_shapes=[
            pltpu.VMEM((_B, 8, _H), jnp.float32),
            pltpu.VMEM((_B, 8, _H), jnp.float32),
            pltpu.VMEM((6, _NWORK), jnp.int32),
            pltpu.SMEM((6, _NWORK), jnp.int32),
            pltpu.SemaphoreType.DMA,
        ],
    )(m1, m2, x1, x2, W, b.reshape(1, _C))
    return out
